# Initial kernel scaffold; baseline (speedup 1.0000x reference)
import functools

import jax
import jax.numpy as jnp
from jax import lax
from jax.experimental import pallas as pl
from jax.experimental.pallas import tpu as pltpu
from jax.experimental.pallas import tpu_sc as plsc

N = 10000
D = 128
E = 320000
NS = 16
EPT = E // NS
K = 100
STEPS = EPT // K
CW = 16


def _sc_body(xu, xi, sui, dui, siu, diu, zf, zc,
             sum_i, cnt_i, sum_u, cnt_u,
             acc, cacc, src_v, dst_v, rows_v, ones_v, sem):
  c = lax.axis_index("c")
  s = lax.axis_index("s")

  @pl.loop(0, K)
  def _(j):
    ones_v[j, :] = jnp.ones((16,), jnp.float32)

  @pl.when(s < NS - 1)
  def _():
    pltpu.sync_copy(zf.at[pl.ds(s * 640, 640)], acc.at[pl.ds(s * 640, 640)])
    pltpu.sync_copy(zc.at[pl.ds(s * 640, 640)], cacc.at[pl.ds(s * 640, 640)])

  @pl.when(s == NS - 1)
  def _():
    pltpu.sync_copy(zf.at[pl.ds(9600, 400)], acc.at[pl.ds(9600, 400)])
    pltpu.sync_copy(zc.at[pl.ds(9600, 400)], cacc.at[pl.ds(9600, 400)])

  plsc.subcore_barrier()

  def run_type(src_hbm, dst_hbm, x_src):
    pltpu.sync_copy(src_hbm.at[s], src_v)
    pltpu.sync_copy(dst_hbm.at[s], dst_v)

    @pl.loop(0, STEPS)
    def _(j):
      pltpu.async_copy(x_src.at[src_v.at[j]], rows_v, sem).wait()
      pltpu.sync_copy(rows_v, acc.at[dst_v.at[j]], add=True)
      pltpu.sync_copy(ones_v, cacc.at[dst_v.at[j]], add=True)

  @pl.when(c == 0)
  def _():
    run_type(sui, dui, xu)

  @pl.when(c == 1)
  def _():
    run_type(siu, diu, xi)

  plsc.subcore_barrier()

  def write_out(sum_o, cnt_o):
    @pl.when(s < NS - 1)
    def _():
      pltpu.sync_copy(acc.at[pl.ds(s * 640, 640)], sum_o.at[pl.ds(s * 640, 640)])
      pltpu.sync_copy(cacc.at[pl.ds(s * 640, 640)], cnt_o.at[pl.ds(s * 640, 640)])

    @pl.when(s == NS - 1)
    def _():
      pltpu.sync_copy(acc.at[pl.ds(9600, 400)], sum_o.at[pl.ds(9600, 400)])
      pltpu.sync_copy(cacc.at[pl.ds(9600, 400)], cnt_o.at[pl.ds(9600, 400)])

  @pl.when(c == 0)
  def _():
    write_out(sum_i, cnt_i)

  @pl.when(c == 1)
  def _():
    write_out(sum_u, cnt_u)


_sc_segment_sums = pl.kernel(
    _sc_body,
    out_type=[
        jax.ShapeDtypeStruct((N, D), jnp.float32),
        jax.ShapeDtypeStruct((N, CW), jnp.float32),
        jax.ShapeDtypeStruct((N, D), jnp.float32),
        jax.ShapeDtypeStruct((N, CW), jnp.float32),
    ],
    mesh=plsc.VectorSubcoreMesh(core_axis_name="c", subcore_axis_name="s"),
    scratch_types=[
        pltpu.VMEM_SHARED((N, D), jnp.float32),
        pltpu.VMEM_SHARED((N, CW), jnp.float32),
        pltpu.VMEM((STEPS, K), jnp.int32),
        pltpu.VMEM((STEPS, K), jnp.int32),
        pltpu.VMEM((K, D), jnp.float32),
        pltpu.VMEM((K, CW), jnp.float32),
        pltpu.SemaphoreType.DMA,
    ],
)


def _tc_body(sum_u, cnt_u, xu, wlT_iu, wrT_iu, bl_iu, g_u, b_u,
             sum_i, cnt_i, xi, wlT_ui, wrT_ui, bl_ui, g_i, b_i,
             out_u, out_i):
  def post(summed, cnt, xd, wlT, wrT, bl, g, b):
    mean = summed / jnp.maximum(cnt[:, 0:1], 1.0)
    y = (jnp.dot(mean, wlT, preferred_element_type=jnp.float32,
                 precision=lax.Precision.HIGHEST)
         + bl
         + jnp.dot(xd, wrT, preferred_element_type=jnp.float32,
                   precision=lax.Precision.HIGHEST))
    mu = jnp.mean(y, axis=-1, keepdims=True)
    var = jnp.mean((y - mu) ** 2, axis=-1, keepdims=True)
    yn = (y - mu) * lax.rsqrt(var + 1e-5) * g + b
    return yn * 0.5 * (1.0 + lax.erf(yn * 0.7071067811865476))

  out_u[...] = post(sum_u[...], cnt_u[...], xu[...],
                    wlT_iu[...], wrT_iu[...], bl_iu[...], g_u[...], b_u[...])
  out_i[...] = post(sum_i[...], cnt_i[...], xi[...],
                    wlT_ui[...], wrT_ui[...], bl_ui[...], g_i[...], b_i[...])


_TC_BLOCK = 1000


def _tc_call(*args):
  row_spec = pl.BlockSpec((_TC_BLOCK, D), lambda i: (i, 0))
  cnt_spec = pl.BlockSpec((_TC_BLOCK, CW), lambda i: (i, 0))
  w_spec = pl.BlockSpec((D, D), lambda i: (0, 0))
  v_spec = pl.BlockSpec((1, D), lambda i: (0, 0))
  per_type = [row_spec, cnt_spec, row_spec, w_spec, w_spec, v_spec, v_spec, v_spec]
  return pl.pallas_call(
      _tc_body,
      grid=(N // _TC_BLOCK,),
      in_specs=per_type + per_type,
      out_specs=[row_spec, row_spec],
      out_shape=[jax.ShapeDtypeStruct((N, D), jnp.float32),
                 jax.ShapeDtypeStruct((N, D), jnp.float32)],
  )(*args)


def kernel(x_user, x_item, edge_ui, edge_iu, Wl_ui, bl_ui, Wr_ui,
           Wl_iu, bl_iu, Wr_iu, g_user, b_user, g_item, b_item):
  sui = edge_ui[0].reshape(NS, STEPS, K)
  dui = edge_ui[1].reshape(NS, STEPS, K)
  siu = edge_iu[0].reshape(NS, STEPS, K)
  diu = edge_iu[1].reshape(NS, STEPS, K)
  zf = jnp.zeros((N, D), jnp.float32)
  zc = jnp.zeros((N, CW), jnp.float32)
  sum_i, cnt_i, sum_u, cnt_u = _sc_segment_sums(
      x_user, x_item, sui, dui, siu, diu, zf, zc)
  out_u, out_i = _tc_call(
      sum_u, cnt_u, x_user, Wl_iu.T, Wr_iu.T, bl_iu.reshape(1, D),
      g_user.reshape(1, D), b_user.reshape(1, D),
      sum_i, cnt_i, x_item, Wl_ui.T, Wr_ui.T, bl_ui.reshape(1, D),
      g_item.reshape(1, D), b_item.reshape(1, D))
  return (out_u, out_i)


# speedup vs baseline: 7.2112x; 7.2112x over previous
import functools

import jax
import jax.numpy as jnp
from jax import lax
from jax.experimental import pallas as pl
from jax.experimental.pallas import tpu as pltpu
from jax.experimental.pallas import tpu_sc as plsc

N = 10000
D = 128
E = 320000
NS = 16
EPT = E // NS
K = 100
STEPS = EPT // K
CH = 40
NCH = STEPS // CH
CW = 16


def _sc_body(xu, xi, sui, dui, siu, diu, zf, zc,
             sum_i, cnt_i, sum_u, cnt_u,
             acc, cacc, src_v, dst_v, rows_v, ones_v, sem):
  c = lax.axis_index("c")
  s = lax.axis_index("s")

  @pl.loop(0, K)
  def _(j):
    ones_v[j, :] = jnp.ones((16,), jnp.float32)

  @pl.when(s < NS - 1)
  def _():
    pltpu.sync_copy(zf.at[pl.ds(s * 640, 640)], acc.at[pl.ds(s * 640, 640)])
    pltpu.sync_copy(zc.at[pl.ds(s * 640, 640)], cacc.at[pl.ds(s * 640, 640)])

  @pl.when(s == NS - 1)
  def _():
    pltpu.sync_copy(zf.at[pl.ds(9600, 400)], acc.at[pl.ds(9600, 400)])
    pltpu.sync_copy(zc.at[pl.ds(9600, 400)], cacc.at[pl.ds(9600, 400)])

  plsc.subcore_barrier()

  def run_type(src_hbm, dst_hbm, x_src):
    @pl.loop(0, NCH)
    def _(ch):
      pltpu.sync_copy(src_hbm.at[s, pl.ds(ch * CH, CH)], src_v)
      pltpu.sync_copy(dst_hbm.at[s, pl.ds(ch * CH, CH)], dst_v)

      @pl.loop(0, CH)
      def _(j):
        pltpu.async_copy(x_src.at[src_v.at[j]], rows_v, sem).wait()
        pltpu.sync_copy(rows_v, acc.at[dst_v.at[j]], add=True)
        pltpu.sync_copy(ones_v, cacc.at[dst_v.at[j]], add=True)

  @pl.when(c == 0)
  def _():
    run_type(sui, dui, xu)

  @pl.when(c == 1)
  def _():
    run_type(siu, diu, xi)

  plsc.subcore_barrier()

  def write_out(sum_o, cnt_o):
    @pl.when(s < NS - 1)
    def _():
      pltpu.sync_copy(acc.at[pl.ds(s * 640, 640)], sum_o.at[pl.ds(s * 640, 640)])
      pltpu.sync_copy(cacc.at[pl.ds(s * 640, 640)], cnt_o.at[pl.ds(s * 640, 640)])

    @pl.when(s == NS - 1)
    def _():
      pltpu.sync_copy(acc.at[pl.ds(9600, 400)], sum_o.at[pl.ds(9600, 400)])
      pltpu.sync_copy(cacc.at[pl.ds(9600, 400)], cnt_o.at[pl.ds(9600, 400)])

  @pl.when(c == 0)
  def _():
    write_out(sum_i, cnt_i)

  @pl.when(c == 1)
  def _():
    write_out(sum_u, cnt_u)


_sc_segment_sums = pl.kernel(
    _sc_body,
    out_type=[
        jax.ShapeDtypeStruct((N, D), jnp.float32),
        jax.ShapeDtypeStruct((N, CW), jnp.float32),
        jax.ShapeDtypeStruct((N, D), jnp.float32),
        jax.ShapeDtypeStruct((N, CW), jnp.float32),
    ],
    mesh=plsc.VectorSubcoreMesh(core_axis_name="c", subcore_axis_name="s"),
    scratch_types=[
        pltpu.VMEM_SHARED((N, D), jnp.float32),
        pltpu.VMEM_SHARED((N, CW), jnp.float32),
        pltpu.VMEM((CH, K), jnp.int32),
        pltpu.VMEM((CH, K), jnp.int32),
        pltpu.VMEM((K, D), jnp.float32),
        pltpu.VMEM((K, CW), jnp.float32),
        pltpu.SemaphoreType.DMA,
    ],
    compiler_params=pltpu.CompilerParams(use_tc_tiling_on_sc=False),
)


def _tc_body(sum_u, cnt_u, xu, wlT_iu, wrT_iu, bl_iu, g_u, b_u,
             sum_i, cnt_i, xi, wlT_ui, wrT_ui, bl_ui, g_i, b_i,
             out_u, out_i):
  def post(summed, cnt, xd, wlT, wrT, bl, g, b):
    mean = summed / jnp.maximum(cnt[:, 0:1], 1.0)
    y = (jnp.dot(mean, wlT, preferred_element_type=jnp.float32,
                 precision=lax.Precision.HIGHEST)
         + bl
         + jnp.dot(xd, wrT, preferred_element_type=jnp.float32,
                   precision=lax.Precision.HIGHEST))
    mu = jnp.mean(y, axis=-1, keepdims=True)
    var = jnp.mean((y - mu) ** 2, axis=-1, keepdims=True)
    yn = (y - mu) * lax.rsqrt(var + 1e-5) * g + b
    return yn * 0.5 * (1.0 + lax.erf(yn * 0.7071067811865476))

  out_u[...] = post(sum_u[...], cnt_u[...], xu[...],
                    wlT_iu[...], wrT_iu[...], bl_iu[...], g_u[...], b_u[...])
  out_i[...] = post(sum_i[...], cnt_i[...], xi[...],
                    wlT_ui[...], wrT_ui[...], bl_ui[...], g_i[...], b_i[...])


_TC_BLOCK = 1000


def _tc_call(*args):
  row_spec = pl.BlockSpec((_TC_BLOCK, D), lambda i: (i, 0))
  cnt_spec = pl.BlockSpec((_TC_BLOCK, CW), lambda i: (i, 0))
  w_spec = pl.BlockSpec((D, D), lambda i: (0, 0))
  v_spec = pl.BlockSpec((1, D), lambda i: (0, 0))
  per_type = [row_spec, cnt_spec, row_spec, w_spec, w_spec, v_spec, v_spec, v_spec]
  return pl.pallas_call(
      _tc_body,
      grid=(N // _TC_BLOCK,),
      in_specs=per_type + per_type,
      out_specs=[row_spec, row_spec],
      out_shape=[jax.ShapeDtypeStruct((N, D), jnp.float32),
                 jax.ShapeDtypeStruct((N, D), jnp.float32)],
  )(*args)


def kernel(x_user, x_item, edge_ui, edge_iu, Wl_ui, bl_ui, Wr_ui,
           Wl_iu, bl_iu, Wr_iu, g_user, b_user, g_item, b_item):
  sui = edge_ui[0].reshape(NS, STEPS, K)
  dui = edge_ui[1].reshape(NS, STEPS, K)
  siu = edge_iu[0].reshape(NS, STEPS, K)
  diu = edge_iu[1].reshape(NS, STEPS, K)
  zf = jnp.zeros((N, D), jnp.float32)
  zc = jnp.zeros((N, CW), jnp.float32)
  sum_i, cnt_i, sum_u, cnt_u = _sc_segment_sums(
      x_user, x_item, sui, dui, siu, diu, zf, zc)
  out_u, out_i = _tc_call(
      sum_u, cnt_u, x_user, Wl_iu.T, Wr_iu.T, bl_iu.reshape(1, D),
      g_user.reshape(1, D), b_user.reshape(1, D),
      sum_i, cnt_i, x_item, Wl_ui.T, Wr_ui.T, bl_ui.reshape(1, D),
      g_item.reshape(1, D), b_item.reshape(1, D))
  return (out_u, out_i)


# speedup vs baseline: 10.7115x; 1.4854x over previous
import functools

import jax
import jax.numpy as jnp
from jax import lax
from jax.experimental import pallas as pl
from jax.experimental.pallas import tpu as pltpu
from jax.experimental.pallas import tpu_sc as plsc

N = 10000
D = 128
E = 320000
NS = 16
EPT = E // NS
K = 100
STEPS = EPT // K
CH = 40
NCH = STEPS // CH
CW = 16


def _sc_body(xu, xi, sui, dui, siu, diu, zf, zc,
             sum_i, cnt_i, sum_u, cnt_u,
             acc, cacc, src_v, dst_v, rows_a, rows_b, ones_v, semA, semB):
  c = lax.axis_index("c")
  s = lax.axis_index("s")

  @pl.loop(0, K)
  def _(j):
    ones_v[j, :] = jnp.ones((16,), jnp.float32)

  @pl.when(s < NS - 1)
  def _():
    pltpu.sync_copy(zf.at[pl.ds(s * 640, 640)], acc.at[pl.ds(s * 640, 640)])
    pltpu.sync_copy(zc.at[pl.ds(s * 640, 640)], cacc.at[pl.ds(s * 640, 640)])

  @pl.when(s == NS - 1)
  def _():
    pltpu.sync_copy(zf.at[pl.ds(9600, 400)], acc.at[pl.ds(9600, 400)])
    pltpu.sync_copy(zc.at[pl.ds(9600, 400)], cacc.at[pl.ds(9600, 400)])

  plsc.subcore_barrier()

  def run_type(src_hbm, dst_hbm, x_src):
    @pl.loop(0, NCH)
    def _(ch):
      pltpu.sync_copy(src_hbm.at[s, pl.ds(ch * CH, CH)], src_v)
      pltpu.sync_copy(dst_hbm.at[s, pl.ds(ch * CH, CH)], dst_v)
      pltpu.async_copy(x_src.at[src_v.at[0]], rows_a, semA)

      @pl.loop(0, CH // 2)
      def _(h):
        j = 2 * h
        pltpu.async_copy(x_src.at[src_v.at[j + 1]], rows_b, semB)
        pltpu.make_async_copy(x_src.at[src_v.at[j]], rows_a, semA).wait()
        pltpu.sync_copy(rows_a, acc.at[dst_v.at[j]], add=True)
        pltpu.sync_copy(ones_v, cacc.at[dst_v.at[j]], add=True)

        @pl.when(j + 2 < CH)
        def _():
          pltpu.async_copy(x_src.at[src_v.at[j + 2]], rows_a, semA)

        pltpu.make_async_copy(x_src.at[src_v.at[j + 1]], rows_b, semB).wait()
        pltpu.sync_copy(rows_b, acc.at[dst_v.at[j + 1]], add=True)
        pltpu.sync_copy(ones_v, cacc.at[dst_v.at[j + 1]], add=True)

  @pl.when(c == 0)
  def _():
    run_type(sui, dui, xu)

  @pl.when(c == 1)
  def _():
    run_type(siu, diu, xi)

  plsc.subcore_barrier()

  def write_out(sum_o, cnt_o):
    @pl.when(s < NS - 1)
    def _():
      pltpu.sync_copy(acc.at[pl.ds(s * 640, 640)], sum_o.at[pl.ds(s * 640, 640)])
      pltpu.sync_copy(cacc.at[pl.ds(s * 640, 640)], cnt_o.at[pl.ds(s * 640, 640)])

    @pl.when(s == NS - 1)
    def _():
      pltpu.sync_copy(acc.at[pl.ds(9600, 400)], sum_o.at[pl.ds(9600, 400)])
      pltpu.sync_copy(cacc.at[pl.ds(9600, 400)], cnt_o.at[pl.ds(9600, 400)])

  @pl.when(c == 0)
  def _():
    write_out(sum_i, cnt_i)

  @pl.when(c == 1)
  def _():
    write_out(sum_u, cnt_u)


_sc_segment_sums = pl.kernel(
    _sc_body,
    out_type=[
        jax.ShapeDtypeStruct((N, D), jnp.float32),
        jax.ShapeDtypeStruct((N, CW), jnp.float32),
        jax.ShapeDtypeStruct((N, D), jnp.float32),
        jax.ShapeDtypeStruct((N, CW), jnp.float32),
    ],
    mesh=plsc.VectorSubcoreMesh(core_axis_name="c", subcore_axis_name="s"),
    scratch_types=[
        pltpu.VMEM_SHARED((N, D), jnp.float32),
        pltpu.VMEM_SHARED((N, CW), jnp.float32),
        pltpu.VMEM((CH, K), jnp.int32),
        pltpu.VMEM((CH, K), jnp.int32),
        pltpu.VMEM((K, D), jnp.float32),
        pltpu.VMEM((K, D), jnp.float32),
        pltpu.VMEM((K, CW), jnp.float32),
        pltpu.SemaphoreType.DMA,
        pltpu.SemaphoreType.DMA,
    ],
    compiler_params=pltpu.CompilerParams(use_tc_tiling_on_sc=False),
)


def _tc_body(sum_u, cnt_u, xu, wlT_iu, wrT_iu, bl_iu, g_u, b_u,
             sum_i, cnt_i, xi, wlT_ui, wrT_ui, bl_ui, g_i, b_i,
             out_u, out_i):
  def post(summed, cnt, xd, wlT, wrT, bl, g, b):
    mean = summed / jnp.maximum(cnt[:, 0:1], 1.0)
    y = (jnp.dot(mean, wlT, preferred_element_type=jnp.float32,
                 precision=lax.Precision.HIGHEST)
         + bl
         + jnp.dot(xd, wrT, preferred_element_type=jnp.float32,
                   precision=lax.Precision.HIGHEST))
    mu = jnp.mean(y, axis=-1, keepdims=True)
    var = jnp.mean((y - mu) ** 2, axis=-1, keepdims=True)
    yn = (y - mu) * lax.rsqrt(var + 1e-5) * g + b
    return yn * 0.5 * (1.0 + lax.erf(yn * 0.7071067811865476))

  out_u[...] = post(sum_u[...], cnt_u[...], xu[...],
                    wlT_iu[...], wrT_iu[...], bl_iu[...], g_u[...], b_u[...])
  out_i[...] = post(sum_i[...], cnt_i[...], xi[...],
                    wlT_ui[...], wrT_ui[...], bl_ui[...], g_i[...], b_i[...])


_TC_BLOCK = 1000


def _tc_call(*args):
  row_spec = pl.BlockSpec((_TC_BLOCK, D), lambda i: (i, 0))
  cnt_spec = pl.BlockSpec((_TC_BLOCK, CW), lambda i: (i, 0))
  w_spec = pl.BlockSpec((D, D), lambda i: (0, 0))
  v_spec = pl.BlockSpec((1, D), lambda i: (0, 0))
  per_type = [row_spec, cnt_spec, row_spec, w_spec, w_spec, v_spec, v_spec, v_spec]
  return pl.pallas_call(
      _tc_body,
      grid=(N // _TC_BLOCK,),
      in_specs=per_type + per_type,
      out_specs=[row_spec, row_spec],
      out_shape=[jax.ShapeDtypeStruct((N, D), jnp.float32),
                 jax.ShapeDtypeStruct((N, D), jnp.float32)],
  )(*args)


def kernel(x_user, x_item, edge_ui, edge_iu, Wl_ui, bl_ui, Wr_ui,
           Wl_iu, bl_iu, Wr_iu, g_user, b_user, g_item, b_item):
  sui = edge_ui[0].reshape(NS, STEPS, K)
  dui = edge_ui[1].reshape(NS, STEPS, K)
  siu = edge_iu[0].reshape(NS, STEPS, K)
  diu = edge_iu[1].reshape(NS, STEPS, K)
  zf = jnp.zeros((N, D), jnp.float32)
  zc = jnp.zeros((N, CW), jnp.float32)
  sum_i, cnt_i, sum_u, cnt_u = _sc_segment_sums(
      x_user, x_item, sui, dui, siu, diu, zf, zc)
  out_u, out_i = _tc_call(
      sum_u, cnt_u, x_user, Wl_iu.T, Wr_iu.T, bl_iu.reshape(1, D),
      g_user.reshape(1, D), b_user.reshape(1, D),
      sum_i, cnt_i, x_item, Wl_ui.T, Wr_ui.T, bl_ui.reshape(1, D),
      g_item.reshape(1, D), b_item.reshape(1, D))
  return (out_u, out_i)


# speedup vs baseline: 11.6042x; 1.0833x over previous
import functools

import jax
import jax.numpy as jnp
from jax import lax
from jax.experimental import pallas as pl
from jax.experimental.pallas import tpu as pltpu
from jax.experimental.pallas import tpu_sc as plsc

N = 10000
D = 128
E = 320000
NS = 16
EPT = E // NS
K = 100
STEPS = EPT // K
CH = 40
NCH = STEPS // CH
CW = 16


def _sc_body(xu, xi, eui, eiu, zf, zc,
             sum_i, cnt_i, sum_u, cnt_u,
             acc, cacc, src_v, dst_v, rows_a, rows_b, ones_v, semA, semB):
  c = lax.axis_index("c")
  s = lax.axis_index("s")

  @pl.loop(0, K)
  def _(j):
    ones_v[j, :] = jnp.ones((16,), jnp.float32)

  @pl.when(s < NS - 1)
  def _():
    pltpu.sync_copy(zf.at[pl.ds(s * 640, 640)], acc.at[pl.ds(s * 640, 640)])
    pltpu.sync_copy(zc.at[pl.ds(s * 640, 640)], cacc.at[pl.ds(s * 640, 640)])

  @pl.when(s == NS - 1)
  def _():
    pltpu.sync_copy(zf.at[pl.ds(9600, 400)], acc.at[pl.ds(9600, 400)])
    pltpu.sync_copy(zc.at[pl.ds(9600, 400)], cacc.at[pl.ds(9600, 400)])

  plsc.subcore_barrier()

  def run_type(e_hbm, x_src):
    @pl.loop(0, NCH)
    def _(ch):
      pltpu.sync_copy(e_hbm.at[0, s, pl.ds(ch * CH, CH)], src_v)
      pltpu.sync_copy(e_hbm.at[1, s, pl.ds(ch * CH, CH)], dst_v)
      pltpu.async_copy(x_src.at[src_v.at[0]], rows_a, semA)

      @pl.loop(0, CH // 2)
      def _(h):
        j = 2 * h
        pltpu.async_copy(x_src.at[src_v.at[j + 1]], rows_b, semB)
        pltpu.make_async_copy(x_src.at[src_v.at[j]], rows_a, semA).wait()
        pltpu.sync_copy(rows_a, acc.at[dst_v.at[j]], add=True)
        pltpu.sync_copy(ones_v, cacc.at[dst_v.at[j]], add=True)

        @pl.when(j + 2 < CH)
        def _():
          pltpu.async_copy(x_src.at[src_v.at[j + 2]], rows_a, semA)

        pltpu.make_async_copy(x_src.at[src_v.at[j + 1]], rows_b, semB).wait()
        pltpu.sync_copy(rows_b, acc.at[dst_v.at[j + 1]], add=True)
        pltpu.sync_copy(ones_v, cacc.at[dst_v.at[j + 1]], add=True)

  @pl.when(c == 0)
  def _():
    run_type(eui, xu)

  @pl.when(c == 1)
  def _():
    run_type(eiu, xi)

  plsc.subcore_barrier()

  def write_out(sum_o, cnt_o):
    @pl.when(s < NS - 1)
    def _():
      pltpu.sync_copy(acc.at[pl.ds(s * 640, 640)], sum_o.at[pl.ds(s * 640, 640)])
      pltpu.sync_copy(cacc.at[pl.ds(s * 640, 640)], cnt_o.at[pl.ds(s * 640, 640)])

    @pl.when(s == NS - 1)
    def _():
      pltpu.sync_copy(acc.at[pl.ds(9600, 400)], sum_o.at[pl.ds(9600, 400)])
      pltpu.sync_copy(cacc.at[pl.ds(9600, 400)], cnt_o.at[pl.ds(9600, 400)])

  @pl.when(c == 0)
  def _():
    write_out(sum_i, cnt_i)

  @pl.when(c == 1)
  def _():
    write_out(sum_u, cnt_u)


_sc_segment_sums = pl.kernel(
    _sc_body,
    out_type=[
        jax.ShapeDtypeStruct((N, D), jnp.float32),
        jax.ShapeDtypeStruct((N, CW), jnp.float32),
        jax.ShapeDtypeStruct((N, D), jnp.float32),
        jax.ShapeDtypeStruct((N, CW), jnp.float32),
    ],
    mesh=plsc.VectorSubcoreMesh(core_axis_name="c", subcore_axis_name="s"),
    scratch_types=[
        pltpu.VMEM_SHARED((N, D), jnp.float32),
        pltpu.VMEM_SHARED((N, CW), jnp.float32),
        pltpu.VMEM((CH, K), jnp.int32),
        pltpu.VMEM((CH, K), jnp.int32),
        pltpu.VMEM((K, D), jnp.float32),
        pltpu.VMEM((K, D), jnp.float32),
        pltpu.VMEM((K, CW), jnp.float32),
        pltpu.SemaphoreType.DMA,
        pltpu.SemaphoreType.DMA,
    ],
    compiler_params=pltpu.CompilerParams(use_tc_tiling_on_sc=False),
)


def _matmul_t(a, w):
  return lax.dot_general(a, w, (((1,), (1,)), ((), ())),
                         preferred_element_type=jnp.float32,
                         precision=lax.Precision.HIGHEST)


def _tc_body(sum_u, cnt_u, xu, wl_iu, wr_iu, bl_iu, g_u, b_u,
             sum_i, cnt_i, xi, wl_ui, wr_ui, bl_ui, g_i, b_i,
             out_u, out_i):
  def post(summed, cnt, xd, wl, wr, bl, g, b):
    mean = summed / jnp.maximum(cnt[:, 0:1], 1.0)
    y = _matmul_t(mean, wl) + bl + _matmul_t(xd, wr)
    mu = jnp.mean(y, axis=-1, keepdims=True)
    var = jnp.mean((y - mu) ** 2, axis=-1, keepdims=True)
    yn = (y - mu) * lax.rsqrt(var + 1e-5) * g + b
    return yn * 0.5 * (1.0 + lax.erf(yn * 0.7071067811865476))

  out_u[...] = post(sum_u[...], cnt_u[...], xu[...],
                    wl_iu[...], wr_iu[...], bl_iu[...], g_u[...], b_u[...])
  out_i[...] = post(sum_i[...], cnt_i[...], xi[...],
                    wl_ui[...], wr_ui[...], bl_ui[...], g_i[...], b_i[...])


_TC_BLOCK = 2000


def _tc_call(*args):
  row_spec = pl.BlockSpec((_TC_BLOCK, D), lambda i: (i, 0))
  cnt_spec = pl.BlockSpec((_TC_BLOCK, CW), lambda i: (i, 0))
  w_spec = pl.BlockSpec((D, D), lambda i: (0, 0))
  v_spec = pl.BlockSpec((1, D), lambda i: (0, 0))
  per_type = [row_spec, cnt_spec, row_spec, w_spec, w_spec, v_spec, v_spec, v_spec]
  return pl.pallas_call(
      _tc_body,
      grid=(N // _TC_BLOCK,),
      in_specs=per_type + per_type,
      out_specs=[row_spec, row_spec],
      out_shape=[jax.ShapeDtypeStruct((N, D), jnp.float32),
                 jax.ShapeDtypeStruct((N, D), jnp.float32)],
  )(*args)


def kernel(x_user, x_item, edge_ui, edge_iu, Wl_ui, bl_ui, Wr_ui,
           Wl_iu, bl_iu, Wr_iu, g_user, b_user, g_item, b_item):
  eui = edge_ui.reshape(2, NS, STEPS, K)
  eiu = edge_iu.reshape(2, NS, STEPS, K)
  zf = jnp.zeros((N, D), jnp.float32)
  zc = jnp.zeros((N, CW), jnp.float32)
  sum_i, cnt_i, sum_u, cnt_u = _sc_segment_sums(
      x_user, x_item, eui, eiu, zf, zc)
  out_u, out_i = _tc_call(
      sum_u, cnt_u, x_user, Wl_iu, Wr_iu, bl_iu.reshape(1, D),
      g_user.reshape(1, D), b_user.reshape(1, D),
      sum_i, cnt_i, x_item, Wl_ui, Wr_ui, bl_ui.reshape(1, D),
      g_item.reshape(1, D), b_item.reshape(1, D))
  return (out_u, out_i)


# speedup vs baseline: 11.7887x; 1.0159x over previous
import functools

import jax
import jax.numpy as jnp
from jax import lax
from jax.experimental import pallas as pl
from jax.experimental.pallas import tpu as pltpu
from jax.experimental.pallas import tpu_sc as plsc

N = 10000
D = 128
E = 320000
NS = 16
EPT = E // NS
K = 100
STEPS = EPT // K
CH = 40
NCH = STEPS // CH
CW = 16


def _sc_body(xu, xi, eui, eiu, zf, zc,
             sum_i, cnt_i, sum_u, cnt_u,
             acc, cacc, src_v, dst_v, rows_a, rows_b, ones_v, semA, semB):
  c = lax.axis_index("c")
  s = lax.axis_index("s")

  @pl.loop(0, K)
  def _(j):
    ones_v[j, :] = jnp.ones((16,), jnp.float32)

  @pl.when(s < NS - 1)
  def _():
    pltpu.sync_copy(zf.at[pl.ds(s * 640, 640)], acc.at[pl.ds(s * 640, 640)])
    pltpu.sync_copy(zc.at[pl.ds(s * 640, 640)], cacc.at[pl.ds(s * 640, 640)])

  @pl.when(s == NS - 1)
  def _():
    pltpu.sync_copy(zf.at[pl.ds(9600, 400)], acc.at[pl.ds(9600, 400)])
    pltpu.sync_copy(zc.at[pl.ds(9600, 400)], cacc.at[pl.ds(9600, 400)])

  plsc.subcore_barrier()

  def run_type(e_hbm, x_src):
    @pl.loop(0, NCH)
    def _(ch):
      pltpu.sync_copy(e_hbm.at[0, s, pl.ds(ch * CH, CH)], src_v)
      pltpu.sync_copy(e_hbm.at[1, s, pl.ds(ch * CH, CH)], dst_v)
      pltpu.async_copy(x_src.at[src_v.at[0]], rows_a, semA)

      @pl.loop(0, CH // 2)
      def _(h):
        j = 2 * h
        pltpu.async_copy(x_src.at[src_v.at[j + 1]], rows_b, semB)
        pltpu.make_async_copy(x_src.at[src_v.at[j]], rows_a, semA).wait()
        pltpu.sync_copy(rows_a, acc.at[dst_v.at[j]], add=True)
        pltpu.sync_copy(ones_v, cacc.at[dst_v.at[j]], add=True)

        @pl.when(j + 2 < CH)
        def _():
          pltpu.async_copy(x_src.at[src_v.at[j + 2]], rows_a, semA)

        pltpu.make_async_copy(x_src.at[src_v.at[j + 1]], rows_b, semB).wait()
        pltpu.sync_copy(rows_b, acc.at[dst_v.at[j + 1]], add=True)
        pltpu.sync_copy(ones_v, cacc.at[dst_v.at[j + 1]], add=True)

  @pl.when(c == 0)
  def _():
    run_type(eui, xu)

  @pl.when(c == 1)
  def _():
    run_type(eiu, xi)

  plsc.subcore_barrier()

  def write_out(sum_o, cnt_o):
    @pl.when(s < NS - 1)
    def _():
      pltpu.sync_copy(acc.at[pl.ds(s * 640, 640)], sum_o.at[pl.ds(s * 640, 640)])
      pltpu.sync_copy(cacc.at[pl.ds(s * 640, 640)], cnt_o.at[pl.ds(s * 640, 640)])

    @pl.when(s == NS - 1)
    def _():
      pltpu.sync_copy(acc.at[pl.ds(9600, 400)], sum_o.at[pl.ds(9600, 400)])
      pltpu.sync_copy(cacc.at[pl.ds(9600, 400)], cnt_o.at[pl.ds(9600, 400)])

  @pl.when(c == 0)
  def _():
    write_out(sum_i, cnt_i)

  @pl.when(c == 1)
  def _():
    write_out(sum_u, cnt_u)


_sc_segment_sums = pl.kernel(
    _sc_body,
    out_type=[
        jax.ShapeDtypeStruct((N, D), jnp.float32),
        jax.ShapeDtypeStruct((N, CW), jnp.float32),
        jax.ShapeDtypeStruct((N, D), jnp.float32),
        jax.ShapeDtypeStruct((N, CW), jnp.float32),
    ],
    mesh=plsc.VectorSubcoreMesh(core_axis_name="c", subcore_axis_name="s"),
    scratch_types=[
        pltpu.VMEM_SHARED((N, D), jnp.float32),
        pltpu.VMEM_SHARED((N, CW), jnp.float32),
        pltpu.VMEM((CH, K), jnp.int32),
        pltpu.VMEM((CH, K), jnp.int32),
        pltpu.VMEM((K, D), jnp.float32),
        pltpu.VMEM((K, D), jnp.float32),
        pltpu.VMEM((K, CW), jnp.float32),
        pltpu.SemaphoreType.DMA,
        pltpu.SemaphoreType.DMA,
    ],
    compiler_params=pltpu.CompilerParams(use_tc_tiling_on_sc=False),
)


def _matmul_t(a, w):
  return lax.dot_general(a, w, (((1,), (1,)), ((), ())),
                         preferred_element_type=jnp.float32,
                         precision=lax.Precision.HIGHEST)


def _tc_pre_body(xu, wr_iu, bl_iu, xi, wr_ui, bl_ui, pre_u, pre_i):
  pre_u[...] = _matmul_t(xu[...], wr_iu[...]) + bl_iu[...]
  pre_i[...] = _matmul_t(xi[...], wr_ui[...]) + bl_ui[...]


def _tc_body(sum_u, cnt_u, pre_u, wl_iu, g_u, b_u,
             sum_i, cnt_i, pre_i, wl_ui, g_i, b_i,
             out_u, out_i):
  def post(summed, cnt, pre, wl, g, b):
    mean = summed / jnp.maximum(cnt[:, 0:1], 1.0)
    y = _matmul_t(mean, wl) + pre
    mu = jnp.mean(y, axis=-1, keepdims=True)
    var = jnp.mean((y - mu) ** 2, axis=-1, keepdims=True)
    yn = (y - mu) * lax.rsqrt(var + 1e-5) * g + b
    return yn * 0.5 * (1.0 + lax.erf(yn * 0.7071067811865476))

  out_u[...] = post(sum_u[...], cnt_u[...], pre_u[...],
                    wl_iu[...], g_u[...], b_u[...])
  out_i[...] = post(sum_i[...], cnt_i[...], pre_i[...],
                    wl_ui[...], g_i[...], b_i[...])


_TC_BLOCK = 2000
_row_spec = pl.BlockSpec((_TC_BLOCK, D), lambda i: (i, 0))
_cnt_spec = pl.BlockSpec((_TC_BLOCK, CW), lambda i: (i, 0))
_w_spec = pl.BlockSpec((D, D), lambda i: (0, 0))
_v_spec = pl.BlockSpec((D,), lambda i: (0,))
_row_out = [jax.ShapeDtypeStruct((N, D), jnp.float32),
            jax.ShapeDtypeStruct((N, D), jnp.float32)]


def _tc_pre(*args):
  per_type = [_row_spec, _w_spec, _v_spec]
  return pl.pallas_call(
      _tc_pre_body,
      grid=(N // _TC_BLOCK,),
      in_specs=per_type + per_type,
      out_specs=[_row_spec, _row_spec],
      out_shape=_row_out,
  )(*args)


def _tc_call(*args):
  per_type = [_row_spec, _cnt_spec, _row_spec, _w_spec, _v_spec, _v_spec]
  return pl.pallas_call(
      _tc_body,
      grid=(N // _TC_BLOCK,),
      in_specs=per_type + per_type,
      out_specs=[_row_spec, _row_spec],
      out_shape=_row_out,
  )(*args)


def kernel(x_user, x_item, edge_ui, edge_iu, Wl_ui, bl_ui, Wr_ui,
           Wl_iu, bl_iu, Wr_iu, g_user, b_user, g_item, b_item):
  eui = edge_ui.reshape(2, NS, STEPS, K)
  eiu = edge_iu.reshape(2, NS, STEPS, K)
  zf = jnp.zeros((N, D), jnp.float32)
  zc = jnp.zeros((N, CW), jnp.float32)
  sum_i, cnt_i, sum_u, cnt_u = _sc_segment_sums(
      x_user, x_item, eui, eiu, zf, zc)
  pre_u, pre_i = _tc_pre(x_user, Wr_iu, bl_iu, x_item, Wr_ui, bl_ui)
  out_u, out_i = _tc_call(
      sum_u, cnt_u, pre_u, Wl_iu, g_user, b_user,
      sum_i, cnt_i, pre_i, Wl_ui, g_item, b_item)
  return (out_u, out_i)
